# Initial kernel scaffold; baseline (speedup 1.0000x reference)
import jax
import jax.numpy as jnp
from jax.experimental import pallas as pl


def kernel(x, edge_index, eps1, W1a, b1a, W1b, b1b, eps2, W2a, b2a, W2b, b2b, Wl, bl):
    raise NotImplementedError("write your pallas kernel here")


# speedup vs baseline: 5.1948x; 5.1948x over previous
import functools

import jax
import jax.numpy as jnp
from jax import lax
from jax.experimental import pallas as pl
from jax.experimental.pallas import tpu as pltpu
from jax.experimental.pallas import tpu_sc as plsc

N = 10000
E = 320000
D = 128
D_OUT = 40

NC = 2
NS = 16
NW = NC * NS
EPW = E // NW
CHUNK = 128
NFULL = EPW // CHUNK
TAIL = EPW - NFULL * CHUNK
RPT = (N // NS) // 8 * 8
REM_R = N - NS * RPT


def _agg_body(feat, src, dst, zeros_hbm, out, idx_s, idx_d, rows,
              idx_s_t, idx_d_t, rows_t, acc, sem):
    c = lax.axis_index("c")
    s = lax.axis_index("s")
    wid = c * NS + s
    base = wid * EPW

    pltpu.sync_copy(zeros_hbm.at[pl.ds(s * RPT, RPT)],
                    acc.at[pl.ds(s * RPT, RPT)])

    @pl.when(s == 0)
    def _():
        pltpu.sync_copy(zeros_hbm.at[pl.ds(NS * RPT, REM_R)],
                        acc.at[pl.ds(NS * RPT, REM_R)])

    plsc.subcore_barrier()

    def ebody(j, carry):
        off = base + j * CHUNK
        pltpu.sync_copy(src.at[pl.ds(off, CHUNK)], idx_s)
        pltpu.sync_copy(dst.at[pl.ds(off, CHUNK)], idx_d)
        pltpu.async_copy(feat.at[idx_s], rows, sem).wait()
        pltpu.sync_copy(rows, acc.at[idx_d], add=True)
        return carry

    lax.fori_loop(0, NFULL, ebody, 0)

    off = base + NFULL * CHUNK
    pltpu.sync_copy(src.at[pl.ds(off, TAIL)], idx_s_t)
    pltpu.sync_copy(dst.at[pl.ds(off, TAIL)], idx_d_t)
    pltpu.async_copy(feat.at[idx_s_t], rows_t, sem).wait()
    pltpu.sync_copy(rows_t, acc.at[idx_d_t], add=True)

    plsc.subcore_barrier()
    pltpu.sync_copy(acc.at[pl.ds(s * RPT, RPT)],
                    out.at[c, pl.ds(s * RPT, RPT)])

    @pl.when(s == 0)
    def _():
        pltpu.sync_copy(acc.at[pl.ds(NS * RPT, REM_R)],
                        out.at[c, pl.ds(NS * RPT, REM_R)])


def _scatter_add(feat, src, dst, zeros_hbm):
    mesh = plsc.VectorSubcoreMesh(core_axis_name="c", subcore_axis_name="s")
    return pl.kernel(
        _agg_body,
        out_type=jax.ShapeDtypeStruct((NC, N, D), jnp.float32),
        mesh=mesh,
        scratch_types=[
            pltpu.VMEM((CHUNK,), jnp.int32),
            pltpu.VMEM((CHUNK,), jnp.int32),
            pltpu.VMEM((CHUNK, D), jnp.float32),
            pltpu.VMEM((TAIL,), jnp.int32),
            pltpu.VMEM((TAIL,), jnp.int32),
            pltpu.VMEM((TAIL, D), jnp.float32),
            pltpu.VMEM_SHARED((N, D), jnp.float32),
            pltpu.SemaphoreType.DMA,
        ],
    )(feat, src, dst, zeros_hbm)


BR = 1000


def _mlp1_body(scale_ref, x_ref, p_ref, wa_ref, ba_ref, wb_ref, bb_ref, o_ref):
    h = x_ref[...] * scale_ref[0] + p_ref[0] + p_ref[1]
    h = jnp.maximum(
        jnp.dot(h, wa_ref[...], preferred_element_type=jnp.float32,
                precision=lax.Precision.HIGHEST) + ba_ref[...], 0.0)
    h = jnp.dot(h, wb_ref[...], preferred_element_type=jnp.float32,
                precision=lax.Precision.HIGHEST) + bb_ref[...]
    o_ref[...] = jnp.maximum(h, 0.0)


def _mlp2_body(scale_ref, x_ref, p_ref, wa_ref, ba_ref, wb_ref, bb_ref,
               wl_ref, bl_ref, o_ref):
    h = x_ref[...] * scale_ref[0] + p_ref[0] + p_ref[1]
    h = jnp.maximum(
        jnp.dot(h, wa_ref[...], preferred_element_type=jnp.float32,
                precision=lax.Precision.HIGHEST) + ba_ref[...], 0.0)
    h = jnp.dot(h, wb_ref[...], preferred_element_type=jnp.float32,
                precision=lax.Precision.HIGHEST) + bb_ref[...]
    h = jnp.maximum(h, 0.0)
    logits = jnp.dot(h, wl_ref[...], preferred_element_type=jnp.float32,
                     precision=lax.Precision.HIGHEST) + bl_ref[...]
    m = jnp.max(logits, axis=1, keepdims=True)
    lse = jnp.log(jnp.sum(jnp.exp(logits - m), axis=1, keepdims=True))
    o_ref[...] = logits - m - lse


def _row_specs():
    full = lambda shape: pl.BlockSpec(shape, lambda i: (0,) * len(shape))
    return full


def _mlp1(scale, x, p, wa, ba, wb, bb):
    full = _row_specs()
    return pl.pallas_call(
        _mlp1_body,
        grid=(N // BR,),
        in_specs=[
            pl.BlockSpec(memory_space=pltpu.SMEM),
            pl.BlockSpec((BR, D), lambda i: (i, 0)),
            pl.BlockSpec((NC, BR, D), lambda i: (0, i, 0)),
            full((D, D)), full((1, D)), full((D, D)), full((1, D)),
        ],
        out_specs=pl.BlockSpec((BR, D), lambda i: (i, 0)),
        out_shape=jax.ShapeDtypeStruct((N, D), jnp.float32),
    )(scale, x, p, wa, ba, wb, bb)


def _mlp2(scale, x, p, wa, ba, wb, bb, wl, bl):
    full = _row_specs()
    return pl.pallas_call(
        _mlp2_body,
        grid=(N // BR,),
        in_specs=[
            pl.BlockSpec(memory_space=pltpu.SMEM),
            pl.BlockSpec((BR, D), lambda i: (i, 0)),
            pl.BlockSpec((NC, BR, D), lambda i: (0, i, 0)),
            full((D, D)), full((1, D)), full((D, D)), full((1, D)),
            full((D, D)), full((1, D)),
        ],
        out_specs=pl.BlockSpec((BR, D), lambda i: (i, 0)),
        out_shape=jax.ShapeDtypeStruct((N, D), jnp.float32),
    )(scale, x, p, wa, ba, wb, bb, wl, bl)


def kernel(x, edge_index, eps1, W1a, b1a, W1b, b1b, eps2, W2a, b2a, W2b, b2b,
           Wl, bl):
    src = edge_index[0].astype(jnp.int32)
    dst = edge_index[1].astype(jnp.int32)
    zeros_hbm = jnp.zeros((N, D), jnp.float32)

    scale1 = jnp.reshape(1.0 + eps1, (1,))
    scale2 = jnp.reshape(1.0 + eps2, (1,))
    b1a_ = jnp.reshape(b1a, (1, D))
    b1b_ = jnp.reshape(b1b, (1, D))
    b2a_ = jnp.reshape(b2a, (1, D))
    b2b_ = jnp.reshape(b2b, (1, D))
    Wl_pad = jnp.pad(Wl, ((0, 0), (0, D - D_OUT)))
    bl_pad = jnp.reshape(
        jnp.pad(bl, (0, D - D_OUT), constant_values=-1e30), (1, D))

    p1 = _scatter_add(x, src, dst, zeros_hbm)
    h1 = _mlp1(scale1, x, p1, W1a, b1a_, W1b, b1b_)
    p2 = _scatter_add(h1, src, dst, zeros_hbm)
    out = _mlp2(scale2, h1, p2, W2a, b2a_, W2b, b2b_, Wl_pad, bl_pad)
    return out[:, :D_OUT]


# speedup vs baseline: 9.2952x; 1.7893x over previous
import functools

import jax
import jax.numpy as jnp
from jax import lax
from jax.experimental import pallas as pl
from jax.experimental.pallas import tpu as pltpu
from jax.experimental.pallas import tpu_sc as plsc

N = 10000
E = 320000
D = 128
D_OUT = 40

NC = 2
NS = 16
NW = NC * NS
CHUNK = 128
CPW = 80
NG = 2
GC = CPW // NG
NCH = NW * CPW
EPAD = NCH * CHUNK
N_ACC = N + 16
RPT = (N // NS) // 8 * 8
REM_R = N - NS * RPT


def _agg_body(feat, srcp, dstp, zeros_hbm, out, sidx, didx, rows0, rows1,
              acc, sem0, sem1):
    c = lax.axis_index("c")
    s = lax.axis_index("s")
    wid = c * NS + s

    pltpu.sync_copy(zeros_hbm.at[pl.ds(s * RPT, RPT)],
                    acc.at[pl.ds(s * RPT, RPT)])

    @pl.when(s == 0)
    def _():
        pltpu.sync_copy(zeros_hbm.at[pl.ds(NS * RPT, N_ACC - NS * RPT)],
                        acc.at[pl.ds(NS * RPT, N_ACC - NS * RPT)])

    plsc.subcore_barrier()

    def fire(j, rbuf, sem):
        pltpu.async_copy(feat.at[sidx.at[j]], rbuf, sem)

    def drain(rbuf, sem):
        pltpu.make_async_copy(feat.at[sidx.at[0]], rbuf, sem).wait()

    def scat(j, rbuf):
        pltpu.sync_copy(rbuf, acc.at[didx.at[j]], add=True)

    @pl.loop(0, NG)
    def _(g):
        base = wid * CPW + g * GC
        pltpu.sync_copy(srcp.at[pl.ds(base, GC)], sidx)
        pltpu.sync_copy(dstp.at[pl.ds(base, GC)], didx)
        fire(0, rows0, sem0)

        @pl.loop(0, GC, step=2)
        def _(j):
            fire(j + 1, rows1, sem1)
            drain(rows0, sem0)
            scat(j, rows0)

            @pl.when(j + 2 < GC)
            def _():
                fire(j + 2, rows0, sem0)

            drain(rows1, sem1)
            scat(j + 1, rows1)

    plsc.subcore_barrier()
    pltpu.sync_copy(acc.at[pl.ds(s * RPT, RPT)],
                    out.at[c, pl.ds(s * RPT, RPT)])

    @pl.when(s == 0)
    def _():
        pltpu.sync_copy(acc.at[pl.ds(NS * RPT, REM_R)],
                        out.at[c, pl.ds(NS * RPT, REM_R)])


def _scatter_add(feat, srcp, dstp, zeros_hbm):
    mesh = plsc.VectorSubcoreMesh(core_axis_name="c", subcore_axis_name="s")
    return pl.kernel(
        _agg_body,
        out_type=jax.ShapeDtypeStruct((NC, N, D), jnp.float32),
        mesh=mesh,
        scratch_types=[
            pltpu.VMEM((GC, CHUNK), jnp.int32),
            pltpu.VMEM((GC, CHUNK), jnp.int32),
            pltpu.VMEM((CHUNK, D), jnp.float32),
            pltpu.VMEM((CHUNK, D), jnp.float32),
            pltpu.VMEM_SHARED((N_ACC, D), jnp.float32),
            pltpu.SemaphoreType.DMA,
            pltpu.SemaphoreType.DMA,
        ],
    )(feat, srcp, dstp, zeros_hbm)


BR = 1000


def _mlp1_body(scale_ref, x_ref, p_ref, wa_ref, ba_ref, wb_ref, bb_ref, o_ref):
    h = x_ref[...] * scale_ref[0] + p_ref[0] + p_ref[1]
    h = jnp.maximum(
        jnp.dot(h, wa_ref[...], preferred_element_type=jnp.float32,
                precision=lax.Precision.HIGHEST) + ba_ref[...], 0.0)
    h = jnp.dot(h, wb_ref[...], preferred_element_type=jnp.float32,
                precision=lax.Precision.HIGHEST) + bb_ref[...]
    o_ref[...] = jnp.maximum(h, 0.0)


def _mlp2_body(scale_ref, x_ref, p_ref, wa_ref, ba_ref, wb_ref, bb_ref,
               wl_ref, bl_ref, o_ref):
    h = x_ref[...] * scale_ref[0] + p_ref[0] + p_ref[1]
    h = jnp.maximum(
        jnp.dot(h, wa_ref[...], preferred_element_type=jnp.float32,
                precision=lax.Precision.HIGHEST) + ba_ref[...], 0.0)
    h = jnp.dot(h, wb_ref[...], preferred_element_type=jnp.float32,
                precision=lax.Precision.HIGHEST) + bb_ref[...]
    h = jnp.maximum(h, 0.0)
    logits = jnp.dot(h, wl_ref[...], preferred_element_type=jnp.float32,
                     precision=lax.Precision.HIGHEST) + bl_ref[...]
    m = jnp.max(logits, axis=1, keepdims=True)
    lse = jnp.log(jnp.sum(jnp.exp(logits - m), axis=1, keepdims=True))
    o_ref[...] = logits - m - lse


def _row_specs():
    full = lambda shape: pl.BlockSpec(shape, lambda i: (0,) * len(shape))
    return full


def _mlp1(scale, x, p, wa, ba, wb, bb):
    full = _row_specs()
    return pl.pallas_call(
        _mlp1_body,
        grid=(N // BR,),
        in_specs=[
            pl.BlockSpec(memory_space=pltpu.SMEM),
            pl.BlockSpec((BR, D), lambda i: (i, 0)),
            pl.BlockSpec((NC, BR, D), lambda i: (0, i, 0)),
            full((D, D)), full((1, D)), full((D, D)), full((1, D)),
        ],
        out_specs=pl.BlockSpec((BR, D), lambda i: (i, 0)),
        out_shape=jax.ShapeDtypeStruct((N, D), jnp.float32),
    )(scale, x, p, wa, ba, wb, bb)


def _mlp2(scale, x, p, wa, ba, wb, bb, wl, bl):
    full = _row_specs()
    return pl.pallas_call(
        _mlp2_body,
        grid=(N // BR,),
        in_specs=[
            pl.BlockSpec(memory_space=pltpu.SMEM),
            pl.BlockSpec((BR, D), lambda i: (i, 0)),
            pl.BlockSpec((NC, BR, D), lambda i: (0, i, 0)),
            full((D, D)), full((1, D)), full((D, D)), full((1, D)),
            full((D, D)), full((1, D)),
        ],
        out_specs=pl.BlockSpec((BR, D), lambda i: (i, 0)),
        out_shape=jax.ShapeDtypeStruct((N, D), jnp.float32),
    )(scale, x, p, wa, ba, wb, bb, wl, bl)


def kernel(x, edge_index, eps1, W1a, b1a, W1b, b1b, eps2, W2a, b2a, W2b, b2b,
           Wl, bl):
    src = edge_index[0].astype(jnp.int32)
    dst = edge_index[1].astype(jnp.int32)
    npad = EPAD - E
    pad_i = jnp.arange(npad, dtype=jnp.int32)
    srcp = jnp.concatenate([src, (pad_i * 131) % N]).reshape(NCH, CHUNK)
    dstp = jnp.concatenate([dst, N + (pad_i % 16)]).reshape(NCH, CHUNK)
    zeros_hbm = jnp.zeros((N_ACC, D), jnp.float32)

    scale1 = jnp.reshape(1.0 + eps1, (1,))
    scale2 = jnp.reshape(1.0 + eps2, (1,))
    b1a_ = jnp.reshape(b1a, (1, D))
    b1b_ = jnp.reshape(b1b, (1, D))
    b2a_ = jnp.reshape(b2a, (1, D))
    b2b_ = jnp.reshape(b2b, (1, D))
    Wl_pad = jnp.pad(Wl, ((0, 0), (0, D - D_OUT)))
    bl_pad = jnp.reshape(
        jnp.pad(bl, (0, D - D_OUT), constant_values=-1e30), (1, D))

    p1 = _scatter_add(x, srcp, dstp, zeros_hbm)
    h1 = _mlp1(scale1, x, p1, W1a, b1a_, W1b, b1b_)
    p2 = _scatter_add(h1, srcp, dstp, zeros_hbm)
    out = _mlp2(scale2, h1, p2, W2a, b2a_, W2b, b2b_, Wl_pad, bl_pad)
    return out[:, :D_OUT]


# speedup vs baseline: 10.9295x; 1.1758x over previous
import functools

import jax
import jax.numpy as jnp
from jax import lax
from jax.experimental import pallas as pl
from jax.experimental.pallas import tpu as pltpu
from jax.experimental.pallas import tpu_sc as plsc

N = 10000
E = 320000
D = 128
D_OUT = 40

NC = 2
NS = 16
NW = NC * NS
CHUNK = 128
CPW = 80
NG = 2
GC = CPW // NG
NCH = NW * CPW
EPAD = NCH * CHUNK
N_ACC = N + 16
RPT = (N // NS) // 8 * 8
REM_R = N - NS * RPT


def _agg_body(feat, srcp, dstp, zeros_hbm, out, sidx, didx, rows0, rows1,
              acc, sem0, sem1):
    c = lax.axis_index("c")
    s = lax.axis_index("s")
    wid = c * NS + s

    pltpu.sync_copy(zeros_hbm.at[pl.ds(s * RPT, RPT)],
                    acc.at[pl.ds(s * RPT, RPT)])

    @pl.when(s == 0)
    def _():
        pltpu.sync_copy(zeros_hbm.at[pl.ds(NS * RPT, N_ACC - NS * RPT)],
                        acc.at[pl.ds(NS * RPT, N_ACC - NS * RPT)])

    plsc.subcore_barrier()

    def fire(j, rbuf, sem):
        pltpu.async_copy(feat.at[sidx.at[j]], rbuf, sem)

    def drain(rbuf, sem):
        pltpu.make_async_copy(feat.at[sidx.at[0]], rbuf, sem).wait()

    def scat(j, rbuf):
        pltpu.sync_copy(rbuf, acc.at[didx.at[j]], add=True)

    @pl.loop(0, NG)
    def _(g):
        base = wid * CPW + g * GC
        pltpu.sync_copy(srcp.at[pl.ds(base, GC)], sidx)
        pltpu.sync_copy(dstp.at[pl.ds(base, GC)], didx)
        fire(0, rows0, sem0)

        @pl.loop(0, GC, step=2)
        def _(j):
            fire(j + 1, rows1, sem1)
            drain(rows0, sem0)
            scat(j, rows0)

            @pl.when(j + 2 < GC)
            def _():
                fire(j + 2, rows0, sem0)

            drain(rows1, sem1)
            scat(j + 1, rows1)

    plsc.subcore_barrier()
    pltpu.sync_copy(acc.at[pl.ds(s * RPT, RPT)],
                    out.at[c, pl.ds(s * RPT, RPT)])

    @pl.when(s == 0)
    def _():
        pltpu.sync_copy(acc.at[pl.ds(NS * RPT, REM_R)],
                        out.at[c, pl.ds(NS * RPT, REM_R)])


def _scatter_add(feat, srcp, dstp, zeros_hbm):
    mesh = plsc.VectorSubcoreMesh(core_axis_name="c", subcore_axis_name="s")
    return pl.kernel(
        _agg_body,
        out_type=jax.ShapeDtypeStruct((NC, N, D), jnp.float32),
        mesh=mesh,
        scratch_types=[
            pltpu.VMEM((GC, CHUNK), jnp.int32),
            pltpu.VMEM((GC, CHUNK), jnp.int32),
            pltpu.VMEM((CHUNK, D), jnp.float32),
            pltpu.VMEM((CHUNK, D), jnp.float32),
            pltpu.VMEM_SHARED((N_ACC, D), jnp.float32),
            pltpu.SemaphoreType.DMA,
            pltpu.SemaphoreType.DMA,
        ],
    )(feat, srcp, dstp, zeros_hbm)


BR = 1000


def _mlp1_body(scale_ref, x_ref, p_ref, wa_ref, ba_ref, wb_ref, bb_ref, o_ref):
    h = x_ref[...] * scale_ref[0] + p_ref[0] + p_ref[1]
    h = jnp.maximum(
        jnp.dot(h, wa_ref[...], preferred_element_type=jnp.float32,
                precision=lax.Precision.DEFAULT) + ba_ref[...], 0.0)
    h = jnp.dot(h, wb_ref[...], preferred_element_type=jnp.float32,
                precision=lax.Precision.DEFAULT) + bb_ref[...]
    o_ref[...] = jnp.maximum(h, 0.0)


def _mlp2_body(scale_ref, x_ref, p_ref, wa_ref, ba_ref, wb_ref, bb_ref,
               wl_ref, bl_ref, o_ref):
    h = x_ref[...] * scale_ref[0] + p_ref[0] + p_ref[1]
    h = jnp.maximum(
        jnp.dot(h, wa_ref[...], preferred_element_type=jnp.float32,
                precision=lax.Precision.DEFAULT) + ba_ref[...], 0.0)
    h = jnp.dot(h, wb_ref[...], preferred_element_type=jnp.float32,
                precision=lax.Precision.DEFAULT) + bb_ref[...]
    h = jnp.maximum(h, 0.0)
    logits = jnp.dot(h, wl_ref[...], preferred_element_type=jnp.float32,
                     precision=lax.Precision.DEFAULT) + bl_ref[...]
    m = jnp.max(logits, axis=1, keepdims=True)
    lse = jnp.log(jnp.sum(jnp.exp(logits - m), axis=1, keepdims=True))
    o_ref[...] = logits - m - lse


def _row_specs():
    full = lambda shape: pl.BlockSpec(shape, lambda i: (0,) * len(shape))
    return full


def _mlp1(scale, x, p, wa, ba, wb, bb):
    full = _row_specs()
    return pl.pallas_call(
        _mlp1_body,
        grid=(N // BR,),
        in_specs=[
            pl.BlockSpec(memory_space=pltpu.SMEM),
            pl.BlockSpec((BR, D), lambda i: (i, 0)),
            pl.BlockSpec((NC, BR, D), lambda i: (0, i, 0)),
            full((D, D)), full((1, D)), full((D, D)), full((1, D)),
        ],
        out_specs=pl.BlockSpec((BR, D), lambda i: (i, 0)),
        out_shape=jax.ShapeDtypeStruct((N, D), jnp.float32),
    )(scale, x, p, wa, ba, wb, bb)


def _mlp2(scale, x, p, wa, ba, wb, bb, wl, bl):
    full = _row_specs()
    return pl.pallas_call(
        _mlp2_body,
        grid=(N // BR,),
        in_specs=[
            pl.BlockSpec(memory_space=pltpu.SMEM),
            pl.BlockSpec((BR, D), lambda i: (i, 0)),
            pl.BlockSpec((NC, BR, D), lambda i: (0, i, 0)),
            full((D, D)), full((1, D)), full((D, D)), full((1, D)),
            full((D, D)), full((1, D)),
        ],
        out_specs=pl.BlockSpec((BR, D), lambda i: (i, 0)),
        out_shape=jax.ShapeDtypeStruct((N, D), jnp.float32),
    )(scale, x, p, wa, ba, wb, bb, wl, bl)


def kernel(x, edge_index, eps1, W1a, b1a, W1b, b1b, eps2, W2a, b2a, W2b, b2b,
           Wl, bl):
    src = edge_index[0].astype(jnp.int32)
    dst = edge_index[1].astype(jnp.int32)
    npad = EPAD - E
    pad_i = jnp.arange(npad, dtype=jnp.int32)
    srcp = jnp.concatenate([src, (pad_i * 131) % N]).reshape(NCH, CHUNK)
    dstp = jnp.concatenate([dst, N + (pad_i % 16)]).reshape(NCH, CHUNK)
    zeros_hbm = jnp.zeros((N_ACC, D), jnp.float32)

    scale1 = jnp.reshape(1.0 + eps1, (1,))
    scale2 = jnp.reshape(1.0 + eps2, (1,))
    b1a_ = jnp.reshape(b1a, (1, D))
    b1b_ = jnp.reshape(b1b, (1, D))
    b2a_ = jnp.reshape(b2a, (1, D))
    b2b_ = jnp.reshape(b2b, (1, D))
    Wl_pad = jnp.pad(Wl, ((0, 0), (0, D - D_OUT)))
    bl_pad = jnp.reshape(
        jnp.pad(bl, (0, D - D_OUT), constant_values=-1e30), (1, D))

    p1 = _scatter_add(x, srcp, dstp, zeros_hbm)
    h1 = _mlp1(scale1, x, p1, W1a, b1a_, W1b, b1b_)
    p2 = _scatter_add(h1, srcp, dstp, zeros_hbm)
    out = _mlp2(scale2, h1, p2, W2a, b2a_, W2b, b2b_, Wl_pad, bl_pad)
    return out[:, :D_OUT]


# speedup vs baseline: 11.3265x; 1.0363x over previous
import functools

import jax
import jax.numpy as jnp
from jax import lax
from jax.experimental import pallas as pl
from jax.experimental.pallas import tpu as pltpu
from jax.experimental.pallas import tpu_sc as plsc

N = 10000
E = 320000
D = 128
D_OUT = 40

NC = 2
NS = 16
NW = NC * NS
CHUNK = 128
NCH = E // CHUNK
CPW = 80
NG = 2
GC = CPW // NG
TAIL_C = NCH - (NW - 1) * CPW
RPT = (N // NS) // 8 * 8
REM_R = N - NS * RPT


def _agg_body(feat, eidx, zeros_hbm, out, sidx, didx, rows0, rows1,
              acc, sem0, sem1):
    c = lax.axis_index("c")
    s = lax.axis_index("s")
    wid = c * NS + s

    pltpu.sync_copy(zeros_hbm.at[pl.ds(s * RPT, RPT)],
                    acc.at[pl.ds(s * RPT, RPT)])

    @pl.when(s == 0)
    def _():
        pltpu.sync_copy(zeros_hbm.at[pl.ds(NS * RPT, REM_R)],
                        acc.at[pl.ds(NS * RPT, REM_R)])

    plsc.subcore_barrier()

    def fire(j, rbuf, sem):
        pltpu.async_copy(feat.at[sidx.at[j]], rbuf, sem)

    def drain(rbuf, sem):
        pltpu.make_async_copy(feat.at[sidx.at[0]], rbuf, sem).wait()

    def scat(j, rbuf):
        pltpu.sync_copy(rbuf, acc.at[didx.at[j]], add=True)

    def stage_and_run(base, n):
        pltpu.sync_copy(eidx.at[0, pl.ds(base, n)], sidx.at[pl.ds(0, n)])
        pltpu.sync_copy(eidx.at[1, pl.ds(base, n)], didx.at[pl.ds(0, n)])
        fire(0, rows0, sem0)

        @pl.loop(0, n, step=2)
        def _(j):
            fire(j + 1, rows1, sem1)
            drain(rows0, sem0)
            scat(j, rows0)

            @pl.when(j + 2 < n)
            def _():
                fire(j + 2, rows0, sem0)

            drain(rows1, sem1)
            scat(j + 1, rows1)

    @pl.when(wid < NW - 1)
    def _():
        @pl.loop(0, NG)
        def _(g):
            stage_and_run(wid * CPW + g * GC, GC)

    @pl.when(wid == NW - 1)
    def _():
        stage_and_run((NW - 1) * CPW, TAIL_C)

    plsc.subcore_barrier()
    pltpu.sync_copy(acc.at[pl.ds(s * RPT, RPT)],
                    out.at[c, pl.ds(s * RPT, RPT)])

    @pl.when(s == 0)
    def _():
        pltpu.sync_copy(acc.at[pl.ds(NS * RPT, REM_R)],
                        out.at[c, pl.ds(NS * RPT, REM_R)])


def _scatter_add(feat, eidx, zeros_hbm):
    mesh = plsc.VectorSubcoreMesh(core_axis_name="c", subcore_axis_name="s")
    return pl.kernel(
        _agg_body,
        out_type=jax.ShapeDtypeStruct((NC, N, D), jnp.float32),
        mesh=mesh,
        scratch_types=[
            pltpu.VMEM((GC, CHUNK), jnp.int32),
            pltpu.VMEM((GC, CHUNK), jnp.int32),
            pltpu.VMEM((CHUNK, D), jnp.float32),
            pltpu.VMEM((CHUNK, D), jnp.float32),
            pltpu.VMEM_SHARED((N, D), jnp.float32),
            pltpu.SemaphoreType.DMA,
            pltpu.SemaphoreType.DMA,
        ],
    )(feat, eidx, zeros_hbm)


BR = 1000


def _mlp1_body(scale_ref, x_ref, p_ref, wa_ref, ba_ref, wb_ref, bb_ref, o_ref):
    h = x_ref[...] * scale_ref[0] + p_ref[0] + p_ref[1]
    h = jnp.maximum(
        jnp.dot(h, wa_ref[...], preferred_element_type=jnp.float32,
                precision=lax.Precision.DEFAULT) + ba_ref[...], 0.0)
    h = jnp.dot(h, wb_ref[...], preferred_element_type=jnp.float32,
                precision=lax.Precision.DEFAULT) + bb_ref[...]
    o_ref[...] = jnp.maximum(h, 0.0)


def _mlp2_body(scale_ref, x_ref, p_ref, wa_ref, ba_ref, wb_ref, bb_ref,
               wl_ref, bl_ref, o_ref):
    h = x_ref[...] * scale_ref[0] + p_ref[0] + p_ref[1]
    h = jnp.maximum(
        jnp.dot(h, wa_ref[...], preferred_element_type=jnp.float32,
                precision=lax.Precision.DEFAULT) + ba_ref[...], 0.0)
    h = jnp.dot(h, wb_ref[...], preferred_element_type=jnp.float32,
                precision=lax.Precision.DEFAULT) + bb_ref[...]
    h = jnp.maximum(h, 0.0)
    logits = jnp.dot(h, wl_ref[...], preferred_element_type=jnp.float32,
                     precision=lax.Precision.DEFAULT) + bl_ref[...]
    m = jnp.max(logits, axis=1, keepdims=True)
    lse = jnp.log(jnp.sum(jnp.exp(logits - m), axis=1, keepdims=True))
    o_ref[...] = logits - m - lse


def _row_specs():
    full = lambda shape: pl.BlockSpec(shape, lambda i: (0,) * len(shape))
    return full


def _mlp1(scale, x, p, wa, ba, wb, bb):
    full = _row_specs()
    return pl.pallas_call(
        _mlp1_body,
        grid=(N // BR,),
        in_specs=[
            pl.BlockSpec(memory_space=pltpu.SMEM),
            pl.BlockSpec((BR, D), lambda i: (i, 0)),
            pl.BlockSpec((NC, BR, D), lambda i: (0, i, 0)),
            full((D, D)), full((1, D)), full((D, D)), full((1, D)),
        ],
        out_specs=pl.BlockSpec((BR, D), lambda i: (i, 0)),
        out_shape=jax.ShapeDtypeStruct((N, D), jnp.float32),
    )(scale, x, p, wa, ba, wb, bb)


def _mlp2(scale, x, p, wa, ba, wb, bb, wl, bl):
    full = _row_specs()
    return pl.pallas_call(
        _mlp2_body,
        grid=(N // BR,),
        in_specs=[
            pl.BlockSpec(memory_space=pltpu.SMEM),
            pl.BlockSpec((BR, D), lambda i: (i, 0)),
            pl.BlockSpec((NC, BR, D), lambda i: (0, i, 0)),
            full((D, D)), full((1, D)), full((D, D)), full((1, D)),
            full((D, D)), full((1, D)),
        ],
        out_specs=pl.BlockSpec((BR, D), lambda i: (i, 0)),
        out_shape=jax.ShapeDtypeStruct((N, D), jnp.float32),
    )(scale, x, p, wa, ba, wb, bb, wl, bl)


def kernel(x, edge_index, eps1, W1a, b1a, W1b, b1b, eps2, W2a, b2a, W2b, b2b,
           Wl, bl):
    eidx = edge_index.astype(jnp.int32).reshape(2, NCH, CHUNK)
    zeros_hbm = jnp.zeros((N, D), jnp.float32)

    scale1 = jnp.reshape(1.0 + eps1, (1,))
    scale2 = jnp.reshape(1.0 + eps2, (1,))
    b1a_ = jnp.reshape(b1a, (1, D))
    b1b_ = jnp.reshape(b1b, (1, D))
    b2a_ = jnp.reshape(b2a, (1, D))
    b2b_ = jnp.reshape(b2b, (1, D))
    Wl_pad = jnp.pad(Wl, ((0, 0), (0, D - D_OUT)))
    bl_pad = jnp.reshape(
        jnp.pad(bl, (0, D - D_OUT), constant_values=-1e30), (1, D))

    p1 = _scatter_add(x, eidx, zeros_hbm)
    h1 = _mlp1(scale1, x, p1, W1a, b1a_, W1b, b1b_)
    p2 = _scatter_add(h1, eidx, zeros_hbm)
    out = _mlp2(scale2, h1, p2, W2a, b2a_, W2b, b2b_, Wl_pad, bl_pad)
    return out[:, :D_OUT]


# speedup vs baseline: 11.7979x; 1.0416x over previous
import functools

import jax
import jax.numpy as jnp
from jax import lax
from jax.experimental import pallas as pl
from jax.experimental.pallas import tpu as pltpu
from jax.experimental.pallas import tpu_sc as plsc

N = 10000
E = 320000
D = 128
D_OUT = 40

NC = 2
NS = 16
NW = NC * NS
CHUNK = 128
NCH = E // CHUNK
CPW = 80
NG = 2
GC = CPW // NG
TAIL_C = NCH - (NW - 1) * CPW
RPT = (N // NS) // 8 * 8
REM_R = N - NS * RPT


def _agg_body(feat, eidx, out, sidx, didx, rows0, rows1, acc, sem0, sem1):
    c = lax.axis_index("c")
    s = lax.axis_index("s")
    wid = c * NS + s

    def stage(base, n):
        pltpu.sync_copy(eidx.at[0, pl.ds(base, n)], sidx.at[pl.ds(0, n)])
        pltpu.sync_copy(eidx.at[1, pl.ds(base, n)], didx.at[pl.ds(0, n)])

    def fire(j, rbuf, sem):
        pltpu.async_copy(feat.at[sidx.at[j]], rbuf, sem)

    def drain(rbuf, sem):
        pltpu.make_async_copy(feat.at[sidx.at[0]], rbuf, sem).wait()

    def scat(j, rbuf):
        pltpu.sync_copy(rbuf, acc.at[didx.at[j]], add=True)

    zv = jnp.zeros((16,), jnp.float32)

    @pl.loop(0, CHUNK)
    def _(r):
        for k in range(D // 16):
            rows0[r, pl.ds(k * 16, 16)] = zv

    @pl.when(wid < NW - 1)
    def _():
        stage(wid * CPW, GC)

    @pl.when(wid == NW - 1)
    def _():
        stage((NW - 1) * CPW, TAIL_C)

    fire(0, rows1, sem1)

    for k in range(4):
        pltpu.sync_copy(rows0, acc.at[pl.ds(s * RPT + k * CHUNK, CHUNK)])
    pltpu.sync_copy(rows0.at[pl.ds(0, RPT - 4 * CHUNK)],
                    acc.at[pl.ds(s * RPT + 4 * CHUNK, RPT - 4 * CHUNK)])

    @pl.when(s == 0)
    def _():
        pltpu.sync_copy(rows0.at[pl.ds(0, REM_R)],
                        acc.at[pl.ds(NS * RPT, REM_R)])

    plsc.subcore_barrier()

    def run_pipeline(n):
        @pl.loop(0, n, step=2)
        def _(j):
            fire(j + 1, rows0, sem0)
            drain(rows1, sem1)
            scat(j, rows1)

            @pl.when(j + 2 < n)
            def _():
                fire(j + 2, rows1, sem1)

            drain(rows0, sem0)
            scat(j + 1, rows0)

    @pl.when(wid < NW - 1)
    def _():
        run_pipeline(GC)
        stage(wid * CPW + GC, GC)
        fire(0, rows1, sem1)
        run_pipeline(GC)

    @pl.when(wid == NW - 1)
    def _():
        run_pipeline(TAIL_C)

    plsc.subcore_barrier()
    pltpu.sync_copy(acc.at[pl.ds(s * RPT, RPT)],
                    out.at[c, pl.ds(s * RPT, RPT)])

    @pl.when(s == 0)
    def _():
        pltpu.sync_copy(acc.at[pl.ds(NS * RPT, REM_R)],
                        out.at[c, pl.ds(NS * RPT, REM_R)])


def _scatter_add(feat, eidx):
    mesh = plsc.VectorSubcoreMesh(core_axis_name="c", subcore_axis_name="s")
    return pl.kernel(
        _agg_body,
        out_type=jax.ShapeDtypeStruct((NC, N, D), jnp.float32),
        mesh=mesh,
        scratch_types=[
            pltpu.VMEM((GC, CHUNK), jnp.int32),
            pltpu.VMEM((GC, CHUNK), jnp.int32),
            pltpu.VMEM((CHUNK, D), jnp.float32),
            pltpu.VMEM((CHUNK, D), jnp.float32),
            pltpu.VMEM_SHARED((N, D), jnp.float32),
            pltpu.SemaphoreType.DMA,
            pltpu.SemaphoreType.DMA,
        ],
    )(feat, eidx)


BR = 1000


def _mlp1_body(scale_ref, x_ref, p_ref, wa_ref, ba_ref, wb_ref, bb_ref, o_ref):
    h = x_ref[...] * scale_ref[0] + p_ref[0] + p_ref[1]
    h = jnp.maximum(
        jnp.dot(h, wa_ref[...], preferred_element_type=jnp.float32,
                precision=lax.Precision.DEFAULT) + ba_ref[...], 0.0)
    h = jnp.dot(h, wb_ref[...], preferred_element_type=jnp.float32,
                precision=lax.Precision.DEFAULT) + bb_ref[...]
    o_ref[...] = jnp.maximum(h, 0.0)


def _mlp2_body(scale_ref, x_ref, p_ref, wa_ref, ba_ref, wb_ref, bb_ref,
               wl_ref, bl_ref, o_ref):
    h = x_ref[...] * scale_ref[0] + p_ref[0] + p_ref[1]
    h = jnp.maximum(
        jnp.dot(h, wa_ref[...], preferred_element_type=jnp.float32,
                precision=lax.Precision.DEFAULT) + ba_ref[...], 0.0)
    h = jnp.dot(h, wb_ref[...], preferred_element_type=jnp.float32,
                precision=lax.Precision.DEFAULT) + bb_ref[...]
    h = jnp.maximum(h, 0.0)
    logits = jnp.dot(h, wl_ref[...], preferred_element_type=jnp.float32,
                     precision=lax.Precision.DEFAULT) + bl_ref[...]
    m = jnp.max(logits, axis=1, keepdims=True)
    lse = jnp.log(jnp.sum(jnp.exp(logits - m), axis=1, keepdims=True))
    o_ref[...] = (logits - m - lse)[:, :D_OUT]


def _row_specs():
    full = lambda shape: pl.BlockSpec(shape, lambda i: (0,) * len(shape))
    return full


def _mlp1(scale, x, p, wa, ba, wb, bb):
    full = _row_specs()
    return pl.pallas_call(
        _mlp1_body,
        grid=(N // BR,),
        in_specs=[
            pl.BlockSpec(memory_space=pltpu.SMEM),
            pl.BlockSpec((BR, D), lambda i: (i, 0)),
            pl.BlockSpec((NC, BR, D), lambda i: (0, i, 0)),
            full((D, D)), full((1, D)), full((D, D)), full((1, D)),
        ],
        out_specs=pl.BlockSpec((BR, D), lambda i: (i, 0)),
        out_shape=jax.ShapeDtypeStruct((N, D), jnp.float32),
    )(scale, x, p, wa, ba, wb, bb)


def _mlp2(scale, x, p, wa, ba, wb, bb, wl, bl):
    full = _row_specs()
    return pl.pallas_call(
        _mlp2_body,
        grid=(N // BR,),
        in_specs=[
            pl.BlockSpec(memory_space=pltpu.SMEM),
            pl.BlockSpec((BR, D), lambda i: (i, 0)),
            pl.BlockSpec((NC, BR, D), lambda i: (0, i, 0)),
            full((D, D)), full((1, D)), full((D, D)), full((1, D)),
            full((D, D)), full((1, D)),
        ],
        out_specs=pl.BlockSpec((BR, D_OUT), lambda i: (i, 0)),
        out_shape=jax.ShapeDtypeStruct((N, D_OUT), jnp.float32),
    )(scale, x, p, wa, ba, wb, bb, wl, bl)


def kernel(x, edge_index, eps1, W1a, b1a, W1b, b1b, eps2, W2a, b2a, W2b, b2b,
           Wl, bl):
    eidx = edge_index.astype(jnp.int32).reshape(2, NCH, CHUNK)

    scale1 = jnp.reshape(1.0 + eps1, (1,))
    scale2 = jnp.reshape(1.0 + eps2, (1,))
    b1a_ = jnp.reshape(b1a, (1, D))
    b1b_ = jnp.reshape(b1b, (1, D))
    b2a_ = jnp.reshape(b2a, (1, D))
    b2b_ = jnp.reshape(b2b, (1, D))
    Wl_pad = jnp.pad(Wl, ((0, 0), (0, D - D_OUT)))
    bl_pad = jnp.reshape(
        jnp.pad(bl, (0, D - D_OUT), constant_values=-1e30), (1, D))

    p1 = _scatter_add(x, eidx)
    h1 = _mlp1(scale1, x, p1, W1a, b1a_, W1b, b1b_)
    p2 = _scatter_add(h1, eidx)
    out = _mlp2(scale2, h1, p2, W2a, b2a_, W2b, b2b_, Wl_pad, bl_pad)
    return out
